# baseline (speedup 1.0000x reference)
import jax
import jax.numpy as jnp
from jax.experimental import pallas as pl
from jax.experimental.pallas import tpu as pltpu

O_N = 10000
NUM_OBJS_P1 = 180
NIMG = 8
EMB = 128
GDIM = 128
HID = 512
NOISE_DIM = 64

KPAD = 184
CROWS = KPAD + NIMG + NOISE_DIM
BLK = 5000
NB = O_N // BLK


def _prep_kernel(objs_ref, oti_ref, obj_emb_ref, gconv_W_ref, gconv_b_ref,
                 att_W_ref, W1a_ref, W1b_ref, W1c_ref, b1_ref, CC_ref):
    objs_l = objs_ref[...]
    oti_l = oti_ref[...]
    ohT_obj = (jax.lax.broadcasted_iota(jnp.int32, (KPAD, O_N), 0)
               == objs_l).astype(jnp.bfloat16)
    ohT_img = (jax.lax.broadcasted_iota(jnp.int32, (NIMG, O_N), 0)
               == oti_l).astype(jnp.bfloat16)
    histT = jax.lax.dot_general(ohT_obj, ohT_img, (((1,), (1,)), ((), ())),
                                preferred_element_type=jnp.float32)
    table_g = jnp.dot(obj_emb_ref[...], gconv_W_ref[...],
                      preferred_element_type=jnp.float32) + gconv_b_ref[...]
    table_a = jnp.dot(table_g, att_W_ref[...],
                      preferred_element_type=jnp.float32)
    counts = jax.lax.dot_general(
        histT, jnp.ones((KPAD, 1), jnp.float32),
        (((0,), (0,)), ((), ())), preferred_element_type=jnp.float32)
    counts = jnp.where(counts > 0.0, counts, 1.0)
    gc = jax.lax.dot_general(
        histT, table_a, (((0,), (0,)), ((), ())),
        preferred_element_type=jnp.float32) / counts
    tg = jnp.tanh(gc)
    sig = jax.nn.sigmoid(jax.lax.dot_general(
        table_g, tg, (((1,), (1,)), ((), ())),
        preferred_element_type=jnp.float32))
    w = histT * sig
    rep = jax.lax.dot_general(
        w, table_g, (((0,), (0,)), ((), ())),
        preferred_element_type=jnp.float32)
    A = jnp.dot(table_g, W1a_ref[...], preferred_element_type=jnp.float32)
    Brep = jnp.dot(rep, W1b_ref[...],
                   preferred_element_type=jnp.float32) + b1_ref[...]
    CC_ref[...] = jnp.concatenate(
        [A, Brep, W1c_ref[...]], axis=0).astype(jnp.bfloat16)


def _main_kernel(objs_ref, oti_ref, noiseT_ref, CC_ref, W2_ref, b2_ref,
                 out_ref):
    objs_l = objs_ref[0]
    oti_l = oti_ref[0]
    ohT_obj = (jax.lax.broadcasted_iota(jnp.int32, (KPAD, BLK), 0)
               == objs_l).astype(jnp.bfloat16)
    ohT_img = (jax.lax.broadcasted_iota(jnp.int32, (NIMG, BLK), 0)
               == oti_l).astype(jnp.bfloat16)
    M = jnp.concatenate([ohT_obj, ohT_img, noiseT_ref[0]], axis=0)
    h = jax.nn.relu(jax.lax.dot_general(
        M, CC_ref[...], (((0,), (0,)), ((), ())),
        preferred_element_type=jnp.float32))
    out_ref[...] = jnp.dot(h.astype(jnp.bfloat16), W2_ref[...],
                           preferred_element_type=jnp.float32) + b2_ref[...]


@jax.jit
def _run(objs, noise, obj_to_img, obj_emb, gconv_W, gconv_b, att_W,
         box_W1, box_b1, box_W2, box_b2):
    objs_r = objs.astype(jnp.int32).reshape(1, O_N)
    oti_r = obj_to_img.astype(jnp.int32).reshape(1, O_N)
    obj_emb_p = jnp.pad(obj_emb, ((0, KPAD - NUM_OBJS_P1), (0, 0)))
    noiseT = noise.astype(jnp.bfloat16).reshape(NB, BLK, NOISE_DIM).swapaxes(1, 2)
    W2_bf = box_W2.astype(jnp.bfloat16)

    def full(shape, idx=None):
        if idx is None:
            idx = tuple(0 for _ in shape)
        return pl.BlockSpec(shape, lambda b, _i=idx: _i)

    CC = pl.pallas_call(
        _prep_kernel,
        grid=(1,),
        in_specs=[
            full((1, O_N)), full((1, O_N)),
            full((KPAD, EMB)), full((EMB, GDIM)), full((1, GDIM)),
            full((GDIM, GDIM)),
            full((GDIM, HID)),
            full((GDIM, HID), (1, 0)),
            full((NOISE_DIM, HID), (4, 0)),
            full((1, HID)),
        ],
        out_specs=full((CROWS, HID)),
        out_shape=jax.ShapeDtypeStruct((CROWS, HID), jnp.bfloat16),
    )(objs_r, oti_r, obj_emb_p, gconv_W, gconv_b.reshape(1, GDIM), att_W,
      box_W1, box_W1, box_W1, box_b1.reshape(1, HID))

    objs_b = objs.astype(jnp.int32).reshape(NB, 1, BLK)
    oti_b = obj_to_img.astype(jnp.int32).reshape(NB, 1, BLK)
    out = pl.pallas_call(
        _main_kernel,
        grid=(NB,),
        in_specs=[
            pl.BlockSpec((1, 1, BLK), lambda b: (b, 0, 0)),
            pl.BlockSpec((1, 1, BLK), lambda b: (b, 0, 0)),
            pl.BlockSpec((1, NOISE_DIM, BLK), lambda b: (b, 0, 0)),
            full((CROWS, HID)), full((HID, 4)), full((1, 4)),
        ],
        out_specs=pl.BlockSpec((BLK, 4), lambda b: (b, 0)),
        out_shape=jax.ShapeDtypeStruct((O_N, 4), jnp.float32),
    )(objs_b, oti_b, noiseT, CC, W2_bf, box_b2.reshape(1, 4))

    return out


def kernel(objs, triples, noise, obj_to_img, obj_emb, pred_emb, gconv_W,
           gconv_b, att_W, box_W1, box_b1, box_W2, box_b2):
    del triples, pred_emb
    return _run(objs, noise, obj_to_img, obj_emb, gconv_W, gconv_b, att_W,
                box_W1, box_b1, box_W2, box_b2)
